# baseline (speedup 1.0000x reference)
import functools

import jax
import jax.numpy as jnp
from jax import lax
from jax.experimental import pallas as pl
from jax.experimental.pallas import tpu as pltpu
from jax.experimental.pallas import tpu_sc as plsc

N = 10000
NP = 10240
K = 8
SPACE = 4
PROP = 32
EMBED = 128


def _mlp(xp, layers, res=None):
    n = len(layers)
    R = 512
    grid = NP // R
    din = xp.shape[1]

    def body(*refs):
        x_ref = refs[0]
        wrefs = refs[1:1 + 2 * n]
        pos = 1 + 2 * n
        r_ref = None
        if res is not None:
            r_ref = refs[pos]
            pos += 1
        o_ref = refs[pos]
        h = x_ref[...]
        for i in range(n):
            W = wrefs[2 * i][...]
            b = wrefs[2 * i + 1][...]
            h = jnp.dot(h.astype(jnp.bfloat16), W,
                        preferred_element_type=jnp.float32) + b
            if i < n - 1:
                h = jnp.where(h > 0, h, jnp.exp(h) - 1.0)
        if r_ref is not None:
            h = h + r_ref[...]
        o_ref[...] = h

    in_specs = [pl.BlockSpec((R, din), lambda i: (i, 0))]
    args = [xp]
    for (W, b) in layers:
        in_specs.append(pl.BlockSpec(W.shape, lambda i: (0, 0)))
        in_specs.append(pl.BlockSpec((1, b.shape[0]), lambda i: (0, 0)))
        args.append(W.astype(jnp.bfloat16))
        args.append(b.reshape(1, -1))
    if res is not None:
        in_specs.append(pl.BlockSpec((R, res.shape[1]), lambda i: (i, 0)))
        args.append(res)
    dout = layers[-1][0].shape[1]
    return pl.pallas_call(
        body,
        grid=(grid,),
        in_specs=in_specs,
        out_specs=pl.BlockSpec((R, dout), lambda i: (i, 0)),
        out_shape=jax.ShapeDtypeStruct((NP, dout), jnp.float32),
    )(*args)


def _proj(cur, p):
    R = 256
    grid = NP // R
    Wsh = jnp.concatenate([p["Ws"], p["Wh"]], axis=1).astype(jnp.bfloat16)
    bsh = jnp.concatenate([p["bs"], p["bh"]]).reshape(1, -1)

    def body(x_ref, wsh_ref, bsh_ref, wo1_ref,
             sT_ref, snT_ref, sn_ref, h_ref, y1_ref):
        xb = x_ref[...].astype(jnp.bfloat16)
        sh = jnp.dot(xb, wsh_ref[...], preferred_element_type=jnp.float32)
        sh = sh + bsh_ref[...]
        s = sh[:, :SPACE]
        h = sh[:, SPACE:SPACE + PROP]
        sn = jnp.sum(s * s, axis=1, keepdims=True)
        sT_ref[...] = s.astype(jnp.bfloat16).T
        snT_ref[...] = sn.T
        sn_ref[...] = sn
        h_ref[...] = h
        y1_ref[...] = jnp.dot(xb, wo1_ref[...],
                              preferred_element_type=jnp.float32)

    out_shapes = (
        jax.ShapeDtypeStruct((SPACE, NP), jnp.bfloat16),
        jax.ShapeDtypeStruct((1, NP), jnp.float32),
        jax.ShapeDtypeStruct((NP, 1), jnp.float32),
        jax.ShapeDtypeStruct((NP, PROP), jnp.float32),
        jax.ShapeDtypeStruct((NP, EMBED), jnp.float32),
    )
    out_specs = (
        pl.BlockSpec((SPACE, R), lambda i: (0, i)),
        pl.BlockSpec((1, R), lambda i: (0, i)),
        pl.BlockSpec((R, 1), lambda i: (i, 0)),
        pl.BlockSpec((R, PROP), lambda i: (i, 0)),
        pl.BlockSpec((R, EMBED), lambda i: (i, 0)),
    )
    return pl.pallas_call(
        body,
        grid=(grid,),
        in_specs=[
            pl.BlockSpec((R, EMBED), lambda i: (i, 0)),
            pl.BlockSpec(Wsh.shape, lambda i: (0, 0)),
            pl.BlockSpec(bsh.shape, lambda i: (0, 0)),
            pl.BlockSpec(p["Wo1"].shape, lambda i: (0, 0)),
        ],
        out_specs=out_specs,
        out_shape=out_shapes,
    )(cur, Wsh, bsh, p["Wo1"].astype(jnp.bfloat16))


_INF = float("inf")
_PADPEN = 1e30


def _knn(sT, sn, snT):
    R = 256
    C = 512
    NT = NP // C
    grid = NP // R

    def body(srow_ref, sT_ref, snr_ref, snc_ref, vd_ref, vi_ref,
             d2_s, vd_s, vi_s):
        s_rows = srow_ref[...]
        snr = snr_ref[...]
        vd_s[...] = jnp.full((R, K), _INF, jnp.float32)
        vi_s[...] = jnp.zeros((R, K), jnp.int32)
        col_iota = lax.broadcasted_iota(jnp.int32, (R, C), 1)
        for t in range(NT):
            s_cols = sT_ref[:, t * C:(t + 1) * C]
            snc = snc_ref[:, t * C:(t + 1) * C]
            prod = lax.dot_general(
                s_rows, s_cols, (((0,), (0,)), ((), ())),
                preferred_element_type=jnp.float32)
            d2 = snr + snc - 2.0 * prod
            if (t + 1) * C > N:
                d2 = jnp.where(col_iota + t * C >= N, _PADPEN, d2)
            d2_s[...] = d2
            cnt = jnp.sum((d2 < vd_s[:, K - 1:K]).astype(jnp.int32),
                          axis=1, keepdims=True)
            nk = jnp.max(jnp.minimum(cnt, K))
            for k in range(K):
                @pl.when(k < nk)
                def _extract():
                    d2c = d2_s[...]
                    m = jnp.min(d2c, axis=1, keepdims=True)
                    am = jnp.min(jnp.where(d2c == m, col_iota, C + 1),
                                 axis=1, keepdims=True)
                    d2_s[...] = jnp.where(col_iota == am, _INF, d2c)
                    gidx = am + t * C
                    vd = vd_s[...]
                    vi = vi_s[...]
                    lt = m < vd
                    vd_sh = jnp.concatenate([m, vd[:, :K - 1]], axis=1)
                    vi_sh = jnp.concatenate([gidx, vi[:, :K - 1]], axis=1)
                    lt_sh = m < vd_sh
                    vd_s[...] = jnp.where(lt, jnp.where(lt_sh, vd_sh, m), vd)
                    vi_s[...] = jnp.where(lt, jnp.where(lt_sh, vi_sh, gidx),
                                          vi)
        vd_ref[...] = vd_s[...]
        vi_ref[...] = vi_s[...]

    return pl.pallas_call(
        body,
        grid=(grid,),
        in_specs=[
            pl.BlockSpec((SPACE, R), lambda i: (0, i)),
            pl.BlockSpec((SPACE, NP), lambda i: (0, 0)),
            pl.BlockSpec((R, 1), lambda i: (i, 0)),
            pl.BlockSpec((1, NP), lambda i: (0, 0)),
        ],
        scratch_shapes=[
            pltpu.VMEM((R, C), jnp.float32),
            pltpu.VMEM((R, K), jnp.float32),
            pltpu.VMEM((R, K), jnp.int32),
        ],
        out_specs=(
            pl.BlockSpec((R, K), lambda i: (i, 0)),
            pl.BlockSpec((R, K), lambda i: (i, 0)),
        ),
        out_shape=(
            jax.ShapeDtypeStruct((NP, K), jnp.float32),
            jax.ShapeDtypeStruct((NP, K), jnp.int32),
        ),
    )(sT, sT, sn, snT)


_NW = 32
_NPW = NP // _NW
_GW = _NPW * K // 128


def _agg(h, idx, d2):
    idx2 = idx.reshape(_NW, _GW, 128)
    d2f = d2.reshape(NP * K)
    mesh = plsc.VectorSubcoreMesh(core_axis_name="c", subcore_axis_name="s")

    @functools.partial(
        pl.kernel, mesh=mesh,
        compiler_params=pltpu.CompilerParams(use_tc_tiling_on_sc=False),
        out_type=jax.ShapeDtypeStruct((NP, 2 * PROP), jnp.float32),
        scratch_types=[
            pltpu.VMEM((_GW, 128), jnp.int32),
            pltpu.VMEM((_NPW * K,), jnp.float32),
            pltpu.VMEM((_NPW * K, PROP), jnp.float32),
            pltpu.VMEM((_NPW, 2 * PROP), jnp.float32),
            pltpu.SemaphoreType.DMA,
        ],
    )
    def k(h_hbm, idx_hbm, d2_hbm, out_hbm, idx_v, w_v, rows_v, out_v, sem):
        wid = lax.axis_index("s") * 2 + lax.axis_index("c")
        pltpu.sync_copy(idx_hbm.at[wid], idx_v)
        pltpu.sync_copy(d2_hbm.at[pl.ds(wid * (_NPW * K), _NPW * K)], w_v)
        copies = [
            pltpu.async_copy(h_hbm.at[idx_v.at[j]],
                             rows_v.at[pl.ds(j * 128, 128)], sem)
            for j in range(_GW)
        ]
        for c in copies:
            c.wait()

        def wbody(i, carry):
            dv = w_v[pl.ds(i * 16, 16)]
            w_v[pl.ds(i * 16, 16)] = jnp.exp(-10.0 * jnp.maximum(dv, 0.0))
            return carry

        lax.fori_loop(0, _NPW * K // 16, wbody, 0)

        def nbody(i2, carry):
            wv = w_v[pl.ds(i2 * 16, 16)]
            for half in range(2):
                i = i2 * 2 + half
                j0 = i * K
                acc0 = jnp.zeros((16,), jnp.float32)
                acc1 = jnp.zeros((16,), jnp.float32)
                mx0 = jnp.full((16,), -_INF, jnp.float32)
                mx1 = jnp.full((16,), -_INF, jnp.float32)
                for kk in range(K):
                    ws = wv[half * K + kk]
                    h0 = rows_v[j0 + kk, pl.ds(0, 16)]
                    h1 = rows_v[j0 + kk, pl.ds(16, 16)]
                    m0 = h0 * ws
                    m1 = h1 * ws
                    acc0 = acc0 + m0
                    acc1 = acc1 + m1
                    mx0 = jnp.maximum(mx0, m0)
                    mx1 = jnp.maximum(mx1, m1)
                out_v[i, pl.ds(0, 16)] = acc0 * 0.125
                out_v[i, pl.ds(16, 16)] = acc1 * 0.125
                out_v[i, pl.ds(32, 16)] = mx0
                out_v[i, pl.ds(48, 16)] = mx1
            return carry

        lax.fori_loop(0, _NPW // 2, nbody, 0)
        pltpu.sync_copy(out_v, out_hbm.at[pl.ds(wid * _NPW, _NPW)])

    return k(h, idx2, d2f)


def _combine(y1, agg, p):
    R = 512
    grid = NP // R
    b2 = p["bo2"].reshape(1, -1)

    def body(y1_ref, agg_ref, w2_ref, b2_ref, o_ref):
        o_ref[...] = (y1_ref[...]
                      + jnp.dot(agg_ref[...].astype(jnp.bfloat16), w2_ref[...],
                                preferred_element_type=jnp.float32)
                      + b2_ref[...])

    return pl.pallas_call(
        body,
        grid=(grid,),
        in_specs=[
            pl.BlockSpec((R, EMBED), lambda i: (i, 0)),
            pl.BlockSpec((R, 2 * PROP), lambda i: (i, 0)),
            pl.BlockSpec(p["Wo2"].shape, lambda i: (0, 0)),
            pl.BlockSpec(b2.shape, lambda i: (0, 0)),
        ],
        out_specs=pl.BlockSpec((R, EMBED), lambda i: (i, 0)),
        out_shape=jax.ShapeDtypeStruct((NP, EMBED), jnp.float32),
    )(y1, agg, p["Wo2"].astype(jnp.bfloat16), b2)


def _conv(cur, p):
    sT, snT, sn, h, y1 = _proj(cur, p)
    d2, idx = _knn(sT, sn, snT)
    agg = _agg(h, idx, d2)
    return _combine(y1, agg, p)


def kernel(x, batch, params):
    xp = jnp.pad(x, ((0, NP - N), (0, 0)))
    emb = _mlp(xp, params["nn0"])

    def chain(convs):
        embs = []
        cur = emb
        for p in convs:
            cur = _conv(cur, p)
            embs.append(cur)
        return embs

    embs_id = chain(params["conv_id"])
    embs_reg = chain(params["conv_reg"])
    emb_id = jnp.concatenate([xp] + embs_id, axis=1)
    emb_reg = jnp.concatenate([xp] + embs_reg, axis=1)
    preds_id = _mlp(emb_id, params["nn_id"])[:N]
    preds_momentum = _mlp(emb_reg, params["nn_reg"], res=xp[:, 1:5])[:N]
    pred_charge = _mlp(emb_reg, params["nn_charge"])[:N]
    return (preds_id, preds_momentum, pred_charge)


# speedup vs baseline: 1.7651x; 1.7651x over previous
import functools

import jax
import jax.numpy as jnp
from jax import lax
from jax.experimental import pallas as pl
from jax.experimental.pallas import tpu as pltpu
from jax.experimental.pallas import tpu_sc as plsc

N = 10000
NP = 10240
K = 8
SPACE = 4
PROP = 32
EMBED = 128


def _mlp(xp, layers, res=None):
    n = len(layers)
    R = 512
    grid = NP // R
    din = xp.shape[1]

    def body(*refs):
        x_ref = refs[0]
        wrefs = refs[1:1 + 2 * n]
        pos = 1 + 2 * n
        r_ref = None
        if res is not None:
            r_ref = refs[pos]
            pos += 1
        o_ref = refs[pos]
        h = x_ref[...]
        for i in range(n):
            W = wrefs[2 * i][...]
            b = wrefs[2 * i + 1][...]
            h = jnp.dot(h.astype(jnp.bfloat16), W,
                        preferred_element_type=jnp.float32) + b
            if i < n - 1:
                h = jnp.where(h > 0, h, jnp.exp(h) - 1.0)
        if r_ref is not None:
            h = h + r_ref[...]
        o_ref[...] = h

    in_specs = [pl.BlockSpec((R, din), lambda i: (i, 0))]
    args = [xp]
    for (W, b) in layers:
        in_specs.append(pl.BlockSpec(W.shape, lambda i: (0, 0)))
        in_specs.append(pl.BlockSpec((1, b.shape[0]), lambda i: (0, 0)))
        args.append(W.astype(jnp.bfloat16))
        args.append(b.reshape(1, -1))
    if res is not None:
        in_specs.append(pl.BlockSpec((R, res.shape[1]), lambda i: (i, 0)))
        args.append(res)
    dout = layers[-1][0].shape[1]
    return pl.pallas_call(
        body,
        grid=(grid,),
        in_specs=in_specs,
        out_specs=pl.BlockSpec((R, dout), lambda i: (i, 0)),
        out_shape=jax.ShapeDtypeStruct((NP, dout), jnp.float32),
    )(*args)


def _proj(cur, p):
    R = 256
    grid = NP // R
    Wsh = jnp.concatenate([p["Ws"], p["Wh"]], axis=1).astype(jnp.bfloat16)
    bsh = jnp.concatenate([p["bs"], p["bh"]]).reshape(1, -1)

    def body(x_ref, wsh_ref, bsh_ref, wo1_ref,
             sT_ref, snT_ref, sn_ref, h_ref, y1_ref):
        xb = x_ref[...].astype(jnp.bfloat16)
        sh = jnp.dot(xb, wsh_ref[...], preferred_element_type=jnp.float32)
        sh = sh + bsh_ref[...]
        s = sh[:, :SPACE]
        h = sh[:, SPACE:SPACE + PROP]
        sn = jnp.sum(s * s, axis=1, keepdims=True)
        sT_ref[...] = s.astype(jnp.bfloat16).T
        snT_ref[...] = sn.T
        sn_ref[...] = sn
        h_ref[...] = h
        y1_ref[...] = jnp.dot(xb, wo1_ref[...],
                              preferred_element_type=jnp.float32)

    out_shapes = (
        jax.ShapeDtypeStruct((SPACE, NP), jnp.bfloat16),
        jax.ShapeDtypeStruct((1, NP), jnp.float32),
        jax.ShapeDtypeStruct((NP, 1), jnp.float32),
        jax.ShapeDtypeStruct((NP, PROP), jnp.float32),
        jax.ShapeDtypeStruct((NP, EMBED), jnp.float32),
    )
    out_specs = (
        pl.BlockSpec((SPACE, R), lambda i: (0, i)),
        pl.BlockSpec((1, R), lambda i: (0, i)),
        pl.BlockSpec((R, 1), lambda i: (i, 0)),
        pl.BlockSpec((R, PROP), lambda i: (i, 0)),
        pl.BlockSpec((R, EMBED), lambda i: (i, 0)),
    )
    return pl.pallas_call(
        body,
        grid=(grid,),
        in_specs=[
            pl.BlockSpec((R, EMBED), lambda i: (i, 0)),
            pl.BlockSpec(Wsh.shape, lambda i: (0, 0)),
            pl.BlockSpec(bsh.shape, lambda i: (0, 0)),
            pl.BlockSpec(p["Wo1"].shape, lambda i: (0, 0)),
        ],
        out_specs=out_specs,
        out_shape=out_shapes,
    )(cur, Wsh, bsh, p["Wo1"].astype(jnp.bfloat16))


_INF = float("inf")
_PADPEN = 1e30


def _knn(sT, sn, snT):
    R = 512
    C = 2048
    NT = NP // C
    grid = NP // R

    def body(srow_ref, sT_ref, snr_ref, snc_ref, vd_ref, vi_ref):
        s_rows = srow_ref[...]
        snr = snr_ref[...]
        vd = jnp.full((R, K), _INF, jnp.float32)
        vi = jnp.zeros((R, K), jnp.int32)
        for t in range(NT):
            s_cols = sT_ref[:, t * C:(t + 1) * C]
            snc = snc_ref[:, t * C:(t + 1) * C]
            prod = lax.dot_general(
                s_rows, s_cols, (((0,), (0,)), ((), ())),
                preferred_element_type=jnp.float32)
            col_iota = lax.broadcasted_iota(jnp.int32, (R, C), 1) + t * C
            d2 = snr + snc - 2.0 * prod
            if (t + 1) * C > N:
                d2 = jnp.where(col_iota >= N, _PADPEN, d2)
            for _ in range(K):
                m = jnp.min(d2, axis=1, keepdims=True)
                am = jnp.min(jnp.where(d2 == m, col_iota, NP + 1),
                             axis=1, keepdims=True)
                d2 = jnp.where(col_iota == am, _INF, d2)
                lt = m < vd
                vd_sh = jnp.concatenate([m, vd[:, :K - 1]], axis=1)
                vi_sh = jnp.concatenate([am, vi[:, :K - 1]], axis=1)
                lt_sh = m < vd_sh
                vd = jnp.where(lt, jnp.where(lt_sh, vd_sh, m), vd)
                vi = jnp.where(lt, jnp.where(lt_sh, vi_sh, am), vi)
        vd_ref[...] = vd
        vi_ref[...] = vi

    return pl.pallas_call(
        body,
        grid=(grid,),
        in_specs=[
            pl.BlockSpec((SPACE, R), lambda i: (0, i)),
            pl.BlockSpec((SPACE, NP), lambda i: (0, 0)),
            pl.BlockSpec((R, 1), lambda i: (i, 0)),
            pl.BlockSpec((1, NP), lambda i: (0, 0)),
        ],
        out_specs=(
            pl.BlockSpec((R, K), lambda i: (i, 0)),
            pl.BlockSpec((R, K), lambda i: (i, 0)),
        ),
        out_shape=(
            jax.ShapeDtypeStruct((NP, K), jnp.float32),
            jax.ShapeDtypeStruct((NP, K), jnp.int32),
        ),
    )(sT, sT, sn, snT)


_NW = 32
_NPW = NP // _NW
_GW = _NPW * K // 128


def _agg(h, idx, d2):
    idx2 = idx.reshape(_NW, _GW, 128)
    d2f = d2.reshape(NP * K)
    mesh = plsc.VectorSubcoreMesh(core_axis_name="c", subcore_axis_name="s")

    @functools.partial(
        pl.kernel, mesh=mesh,
        compiler_params=pltpu.CompilerParams(use_tc_tiling_on_sc=False),
        out_type=jax.ShapeDtypeStruct((NP, 2 * PROP), jnp.float32),
        scratch_types=[
            pltpu.VMEM((_GW, 128), jnp.int32),
            pltpu.VMEM((_NPW * K,), jnp.float32),
            pltpu.VMEM((_NPW * K, PROP), jnp.float32),
            pltpu.VMEM((_NPW, 2 * PROP), jnp.float32),
            pltpu.SemaphoreType.DMA,
        ],
    )
    def k(h_hbm, idx_hbm, d2_hbm, out_hbm, idx_v, w_v, rows_v, out_v, sem):
        wid = lax.axis_index("s") * 2 + lax.axis_index("c")
        pltpu.sync_copy(idx_hbm.at[wid], idx_v)
        pltpu.sync_copy(d2_hbm.at[pl.ds(wid * (_NPW * K), _NPW * K)], w_v)
        copies = [
            pltpu.async_copy(h_hbm.at[idx_v.at[j]],
                             rows_v.at[pl.ds(j * 128, 128)], sem)
            for j in range(_GW)
        ]
        for c in copies:
            c.wait()

        def wbody(i, carry):
            dv = w_v[pl.ds(i * 16, 16)]
            w_v[pl.ds(i * 16, 16)] = jnp.exp(-10.0 * jnp.maximum(dv, 0.0))
            return carry

        lax.fori_loop(0, _NPW * K // 16, wbody, 0)

        def nbody(i2, carry):
            wv = w_v[pl.ds(i2 * 16, 16)]
            for half in range(2):
                i = i2 * 2 + half
                j0 = i * K
                acc0 = jnp.zeros((16,), jnp.float32)
                acc1 = jnp.zeros((16,), jnp.float32)
                mx0 = jnp.full((16,), -_INF, jnp.float32)
                mx1 = jnp.full((16,), -_INF, jnp.float32)
                for kk in range(K):
                    ws = wv[half * K + kk]
                    h0 = rows_v[j0 + kk, pl.ds(0, 16)]
                    h1 = rows_v[j0 + kk, pl.ds(16, 16)]
                    m0 = h0 * ws
                    m1 = h1 * ws
                    acc0 = acc0 + m0
                    acc1 = acc1 + m1
                    mx0 = jnp.maximum(mx0, m0)
                    mx1 = jnp.maximum(mx1, m1)
                out_v[i, pl.ds(0, 16)] = acc0 * 0.125
                out_v[i, pl.ds(16, 16)] = acc1 * 0.125
                out_v[i, pl.ds(32, 16)] = mx0
                out_v[i, pl.ds(48, 16)] = mx1
            return carry

        lax.fori_loop(0, _NPW // 2, nbody, 0)
        pltpu.sync_copy(out_v, out_hbm.at[pl.ds(wid * _NPW, _NPW)])

    return k(h, idx2, d2f)


def _combine(y1, agg, p):
    R = 512
    grid = NP // R
    b2 = p["bo2"].reshape(1, -1)

    def body(y1_ref, agg_ref, w2_ref, b2_ref, o_ref):
        o_ref[...] = (y1_ref[...]
                      + jnp.dot(agg_ref[...].astype(jnp.bfloat16), w2_ref[...],
                                preferred_element_type=jnp.float32)
                      + b2_ref[...])

    return pl.pallas_call(
        body,
        grid=(grid,),
        in_specs=[
            pl.BlockSpec((R, EMBED), lambda i: (i, 0)),
            pl.BlockSpec((R, 2 * PROP), lambda i: (i, 0)),
            pl.BlockSpec(p["Wo2"].shape, lambda i: (0, 0)),
            pl.BlockSpec(b2.shape, lambda i: (0, 0)),
        ],
        out_specs=pl.BlockSpec((R, EMBED), lambda i: (i, 0)),
        out_shape=jax.ShapeDtypeStruct((NP, EMBED), jnp.float32),
    )(y1, agg, p["Wo2"].astype(jnp.bfloat16), b2)


def _conv(cur, p):
    sT, snT, sn, h, y1 = _proj(cur, p)
    d2, idx = _knn(sT, sn, snT)
    agg = _agg(h, idx, d2)
    return _combine(y1, agg, p)


def kernel(x, batch, params):
    xp = jnp.pad(x, ((0, NP - N), (0, 0)))
    emb = _mlp(xp, params["nn0"])

    def chain(convs):
        embs = []
        cur = emb
        for p in convs:
            cur = _conv(cur, p)
            embs.append(cur)
        return embs

    embs_id = chain(params["conv_id"])
    embs_reg = chain(params["conv_reg"])
    emb_id = jnp.concatenate([xp] + embs_id, axis=1)
    emb_reg = jnp.concatenate([xp] + embs_reg, axis=1)
    preds_id = _mlp(emb_id, params["nn_id"])[:N]
    preds_momentum = _mlp(emb_reg, params["nn_reg"], res=xp[:, 1:5])[:N]
    pred_charge = _mlp(emb_reg, params["nn_charge"])[:N]
    return (preds_id, preds_momentum, pred_charge)


# speedup vs baseline: 2.2176x; 1.2563x over previous
import functools

import jax
import jax.numpy as jnp
from jax import lax
from jax.experimental import pallas as pl
from jax.experimental.pallas import tpu as pltpu
from jax.experimental.pallas import tpu_sc as plsc

N = 10000
NP = 10240
K = 8
SPACE = 4
PROP = 32
EMBED = 128


def _mlp(xp, layers, res=None):
    n = len(layers)
    R = 512
    grid = NP // R
    din = xp.shape[1]

    def body(*refs):
        x_ref = refs[0]
        wrefs = refs[1:1 + 2 * n]
        pos = 1 + 2 * n
        r_ref = None
        if res is not None:
            r_ref = refs[pos]
            pos += 1
        o_ref = refs[pos]
        h = x_ref[...]
        for i in range(n):
            W = wrefs[2 * i][...]
            b = wrefs[2 * i + 1][...]
            h = jnp.dot(h.astype(jnp.bfloat16), W,
                        preferred_element_type=jnp.float32) + b
            if i < n - 1:
                h = jnp.where(h > 0, h, jnp.exp(h) - 1.0)
        if r_ref is not None:
            h = h + r_ref[...]
        o_ref[...] = h

    in_specs = [pl.BlockSpec((R, din), lambda i: (i, 0))]
    args = [xp]
    for (W, b) in layers:
        in_specs.append(pl.BlockSpec(W.shape, lambda i: (0, 0)))
        in_specs.append(pl.BlockSpec((1, b.shape[0]), lambda i: (0, 0)))
        args.append(W.astype(jnp.bfloat16))
        args.append(b.reshape(1, -1))
    if res is not None:
        in_specs.append(pl.BlockSpec((R, res.shape[1]), lambda i: (i, 0)))
        args.append(res)
    dout = layers[-1][0].shape[1]
    return pl.pallas_call(
        body,
        grid=(grid,),
        in_specs=in_specs,
        out_specs=pl.BlockSpec((R, dout), lambda i: (i, 0)),
        out_shape=jax.ShapeDtypeStruct((NP, dout), jnp.float32),
    )(*args)


def _proj(cur, p):
    R = 256
    grid = NP // R
    Wsh = jnp.concatenate([p["Ws"], p["Wh"]], axis=1).astype(jnp.bfloat16)
    bsh = jnp.concatenate([p["bs"], p["bh"]]).reshape(1, -1)

    def body(x_ref, wsh_ref, bsh_ref, wo1_ref,
             sT_ref, snT_ref, sn_ref, h_ref, y1_ref):
        xb = x_ref[...].astype(jnp.bfloat16)
        sh = jnp.dot(xb, wsh_ref[...], preferred_element_type=jnp.float32)
        sh = sh + bsh_ref[...]
        s = sh[:, :SPACE]
        h = sh[:, SPACE:SPACE + PROP]
        sn = jnp.sum(s * s, axis=1, keepdims=True)
        sT_ref[...] = s.astype(jnp.bfloat16).T
        snT_ref[...] = sn.T
        sn_ref[...] = sn
        h_ref[...] = h
        y1_ref[...] = jnp.dot(xb, wo1_ref[...],
                              preferred_element_type=jnp.float32)

    out_shapes = (
        jax.ShapeDtypeStruct((SPACE, NP), jnp.bfloat16),
        jax.ShapeDtypeStruct((1, NP), jnp.float32),
        jax.ShapeDtypeStruct((NP, 1), jnp.float32),
        jax.ShapeDtypeStruct((NP, PROP), jnp.float32),
        jax.ShapeDtypeStruct((NP, EMBED), jnp.float32),
    )
    out_specs = (
        pl.BlockSpec((SPACE, R), lambda i: (0, i)),
        pl.BlockSpec((1, R), lambda i: (0, i)),
        pl.BlockSpec((R, 1), lambda i: (i, 0)),
        pl.BlockSpec((R, PROP), lambda i: (i, 0)),
        pl.BlockSpec((R, EMBED), lambda i: (i, 0)),
    )
    return pl.pallas_call(
        body,
        grid=(grid,),
        in_specs=[
            pl.BlockSpec((R, EMBED), lambda i: (i, 0)),
            pl.BlockSpec(Wsh.shape, lambda i: (0, 0)),
            pl.BlockSpec(bsh.shape, lambda i: (0, 0)),
            pl.BlockSpec(p["Wo1"].shape, lambda i: (0, 0)),
        ],
        out_specs=out_specs,
        out_shape=out_shapes,
    )(cur, Wsh, bsh, p["Wo1"].astype(jnp.bfloat16))


_INF = float("inf")
_PADPEN = 1e30


def _knn(sT, sn, snT):
    R = 256
    C = 2048
    NT = NP // C
    grid = NP // R

    def body(srow_ref, sT_ref, snr_ref, snc_ref, vd_ref, vi_ref):
        s_rows = srow_ref[...].T
        snr = snr_ref[...]
        vd = jnp.full((R, K), _INF, jnp.float32)
        vi = jnp.zeros((R, K), jnp.int32)
        for t in range(NT):
            s_cols = sT_ref[:, t * C:(t + 1) * C]
            snc = snc_ref[:, t * C:(t + 1) * C]
            prod = lax.dot_general(
                s_rows, s_cols, (((1,), (0,)), ((), ())),
                preferred_element_type=jnp.float32)
            col_iota = lax.broadcasted_iota(jnp.int32, (R, C), 1) + t * C
            d2 = snr + snc - 2.0 * prod
            if (t + 1) * C > N:
                d2 = jnp.where(col_iota >= N, _PADPEN, d2)
            for _ in range(K):
                m = jnp.min(d2, axis=1, keepdims=True)
                am = jnp.min(jnp.where(d2 == m, col_iota, NP + 1),
                             axis=1, keepdims=True)
                d2 = jnp.where(col_iota == am, _INF, d2)
                lt = m < vd
                vd_sh = jnp.concatenate([m, vd[:, :K - 1]], axis=1)
                vi_sh = jnp.concatenate([am, vi[:, :K - 1]], axis=1)
                lt_sh = m < vd_sh
                vd = jnp.where(lt, jnp.where(lt_sh, vd_sh, m), vd)
                vi = jnp.where(lt, jnp.where(lt_sh, vi_sh, am), vi)
        vd_ref[...] = vd
        vi_ref[...] = vi

    return pl.pallas_call(
        body,
        grid=(grid,),
        in_specs=[
            pl.BlockSpec((SPACE, R), lambda i: (0, i)),
            pl.BlockSpec((SPACE, NP), lambda i: (0, 0)),
            pl.BlockSpec((R, 1), lambda i: (i, 0)),
            pl.BlockSpec((1, NP), lambda i: (0, 0)),
        ],
        out_specs=(
            pl.BlockSpec((R, K), lambda i: (i, 0)),
            pl.BlockSpec((R, K), lambda i: (i, 0)),
        ),
        out_shape=(
            jax.ShapeDtypeStruct((NP, K), jnp.float32),
            jax.ShapeDtypeStruct((NP, K), jnp.int32),
        ),
    )(sT, sT, sn, snT)


_NW = 32
_NPW = NP // _NW
_GW = _NPW * K // 128


def _agg(h, idx, d2):
    idx2 = idx.reshape(_NW, _GW, 128)
    d2f = d2.reshape(NP * K)
    mesh = plsc.VectorSubcoreMesh(core_axis_name="c", subcore_axis_name="s")

    @functools.partial(
        pl.kernel, mesh=mesh,
        compiler_params=pltpu.CompilerParams(use_tc_tiling_on_sc=False),
        out_type=jax.ShapeDtypeStruct((NP, 2 * PROP), jnp.float32),
        scratch_types=[
            pltpu.VMEM((_GW, 128), jnp.int32),
            pltpu.VMEM((_NPW * K,), jnp.float32),
            pltpu.VMEM((_NPW * K, PROP), jnp.float32),
            pltpu.VMEM((_NPW, 2 * PROP), jnp.float32),
            pltpu.SemaphoreType.DMA,
        ],
    )
    def k(h_hbm, idx_hbm, d2_hbm, out_hbm, idx_v, w_v, rows_v, out_v, sem):
        wid = lax.axis_index("s") * 2 + lax.axis_index("c")
        pltpu.sync_copy(idx_hbm.at[wid], idx_v)
        pltpu.sync_copy(d2_hbm.at[pl.ds(wid * (_NPW * K), _NPW * K)], w_v)
        copies = [
            pltpu.async_copy(h_hbm.at[idx_v.at[j]],
                             rows_v.at[pl.ds(j * 128, 128)], sem)
            for j in range(_GW)
        ]
        for c in copies:
            c.wait()

        def wbody(i, carry):
            dv = w_v[pl.ds(i * 16, 16)]
            w_v[pl.ds(i * 16, 16)] = jnp.exp(-10.0 * jnp.maximum(dv, 0.0))
            return carry

        lax.fori_loop(0, _NPW * K // 16, wbody, 0)

        def nbody(i2, carry):
            wv = w_v[pl.ds(i2 * 16, 16)]
            for half in range(2):
                i = i2 * 2 + half
                j0 = i * K
                acc0 = jnp.zeros((16,), jnp.float32)
                acc1 = jnp.zeros((16,), jnp.float32)
                mx0 = jnp.full((16,), -_INF, jnp.float32)
                mx1 = jnp.full((16,), -_INF, jnp.float32)
                for kk in range(K):
                    ws = wv[half * K + kk]
                    h0 = rows_v[j0 + kk, pl.ds(0, 16)]
                    h1 = rows_v[j0 + kk, pl.ds(16, 16)]
                    m0 = h0 * ws
                    m1 = h1 * ws
                    acc0 = acc0 + m0
                    acc1 = acc1 + m1
                    mx0 = jnp.maximum(mx0, m0)
                    mx1 = jnp.maximum(mx1, m1)
                out_v[i, pl.ds(0, 16)] = acc0 * 0.125
                out_v[i, pl.ds(16, 16)] = acc1 * 0.125
                out_v[i, pl.ds(32, 16)] = mx0
                out_v[i, pl.ds(48, 16)] = mx1
            return carry

        lax.fori_loop(0, _NPW // 2, nbody, 0)
        pltpu.sync_copy(out_v, out_hbm.at[pl.ds(wid * _NPW, _NPW)])

    return k(h, idx2, d2f)


def _combine(y1, agg, p):
    R = 512
    grid = NP // R
    b2 = p["bo2"].reshape(1, -1)

    def body(y1_ref, agg_ref, w2_ref, b2_ref, o_ref):
        o_ref[...] = (y1_ref[...]
                      + jnp.dot(agg_ref[...].astype(jnp.bfloat16), w2_ref[...],
                                preferred_element_type=jnp.float32)
                      + b2_ref[...])

    return pl.pallas_call(
        body,
        grid=(grid,),
        in_specs=[
            pl.BlockSpec((R, EMBED), lambda i: (i, 0)),
            pl.BlockSpec((R, 2 * PROP), lambda i: (i, 0)),
            pl.BlockSpec(p["Wo2"].shape, lambda i: (0, 0)),
            pl.BlockSpec(b2.shape, lambda i: (0, 0)),
        ],
        out_specs=pl.BlockSpec((R, EMBED), lambda i: (i, 0)),
        out_shape=jax.ShapeDtypeStruct((NP, EMBED), jnp.float32),
    )(y1, agg, p["Wo2"].astype(jnp.bfloat16), b2)


def _dotx(a, b):
    return jnp.dot(a.astype(jnp.bfloat16), b.astype(jnp.bfloat16),
                   preferred_element_type=jnp.float32)


def _mlp_x(layers, x):
    n = len(layers)
    for i, (W, b) in enumerate(layers):
        x = _dotx(x, W) + b
        if i < n - 1:
            x = jnp.where(x > 0, x, jnp.expm1(x))
    return x


def _conv(cur, p):
    s = _dotx(cur, p["Ws"]) + p["bs"]
    h = _dotx(cur, p["Wh"]) + p["bh"]
    sn = jnp.sum(s * s, axis=1)
    sbf = s.astype(jnp.bfloat16)
    sT = jnp.pad(sbf, ((0, NP - N), (0, 0))).T
    snp = jnp.pad(sn, (0, NP - N))
    d2, idx = _knn(sT, snp.reshape(NP, 1), snp.reshape(1, NP))
    hp = jnp.pad(h, ((0, NP - N), (0, 0)))
    agg = _agg(hp, idx, d2)[:N]
    return _dotx(cur, p["Wo1"]) + (_dotx(agg, p["Wo2"]) + p["bo2"])


def kernel(x, batch, params):
    emb = _mlp_x(params["nn0"], x)

    def chain(convs):
        embs = []
        cur = emb
        for p in convs:
            cur = _conv(cur, p)
            embs.append(cur)
        return embs

    embs_id = chain(params["conv_id"])
    embs_reg = chain(params["conv_reg"])
    emb_id = jnp.concatenate([x] + embs_id, axis=1)
    emb_reg = jnp.concatenate([x] + embs_reg, axis=1)
    preds_id = _mlp_x(params["nn_id"], emb_id)
    preds_momentum = _mlp_x(params["nn_reg"], emb_reg) + x[:, 1:5]
    pred_charge = _mlp_x(params["nn_charge"], emb_reg)
    return (preds_id, preds_momentum, pred_charge)
